# Initial kernel scaffold; baseline (speedup 1.0000x reference)
import jax
import jax.numpy as jnp
from jax.experimental import pallas as pl


def kernel(source, masked_pos, conv_w0, conv_w1, conv_w2, conv_w3, conv_w4, conv_w5, conv_w6, gn_scale, gn_bias, ln_scale, ln_bias, proj_w, proj_b, mask_emb, projq_w, projq_b, final_w, final_b):
    raise NotImplementedError("write your pallas kernel here")


# speedup vs baseline: 2.1505x; 2.1505x over previous
import functools

import jax
import jax.numpy as jnp
import numpy as np
from jax import lax
from jax.experimental import pallas as pl
from jax.experimental.pallas import tpu as pltpu
from jax.experimental.pallas import tpu_sc as plsc

_B = 2
_M = 60
_NNEG = 100
_TEMP = 0.1
_NOUT = (1 + _NNEG) * _B * _M
_NPAD = 12288


def _neg_index_pattern():
    tszs = jnp.repeat(jnp.arange(_M), _NNEG)
    neg = jax.random.randint(jax.random.key(42), (_B, _NNEG * _M), 0, _M - 1)
    neg = jnp.where(neg >= tszs[None, :], neg + 1, neg)
    neg = neg + jnp.arange(_B)[:, None] * 64
    negpart = jnp.transpose(neg.reshape(_B, _M, _NNEG), (2, 0, 1)).reshape(-1)
    ar = jnp.arange(_B * _M)
    pos = (ar // _M) * 64 + ar % _M
    flat = jnp.concatenate([pos, negpart])
    return jnp.pad(flat, (0, _NPAD - _NOUT)).astype(jnp.int32)


def _gelu(x):
    return jax.nn.gelu(x)


def _k1_body(p_ref, w_ref, s_ref, b_ref, o_ref):
    f32 = jnp.float32
    w = w_ref[...]
    s = jnp.zeros((1, 512), f32)
    ss = jnp.zeros((1, 512), f32)
    for t in range(10):
        x = jnp.dot(p_ref[0, pl.ds(t * 800, 800)], w,
                    preferred_element_type=f32)
        o_ref[0, pl.ds(t * 800, 800)] = x
        s = s + jnp.sum(x, axis=0, keepdims=True)
        ss = ss + jnp.sum(x * x, axis=0, keepdims=True)
    last = o_ref[0, pl.ds(7999, 1)]
    s = s - last
    ss = ss - last * last
    m = s / 7999.0
    v = ss / 7999.0 - m * m
    sc = lax.rsqrt(v + 1e-5) * s_ref[...]
    for t in range(10):
        x = o_ref[0, pl.ds(t * 800, 800)]
        o_ref[0, pl.ds(t * 800, 800)] = _gelu((x - m) * sc + b_ref[...])


def _conv_s2(x, w, n_out, ksize):
    xr = x.reshape(x.shape[0] // 2, 2, 512)
    e = xr[:, 0, :]
    o = xr[:, 1, :]
    acc = jnp.dot(e[:n_out], w[0], preferred_element_type=jnp.float32)
    acc = acc + jnp.dot(o[:n_out], w[1], preferred_element_type=jnp.float32)
    if ksize == 3:
        e1 = jnp.concatenate([e[1:], jnp.zeros((1, 512), jnp.float32)], axis=0)
        acc = acc + jnp.dot(e1[:n_out], w[2], preferred_element_type=jnp.float32)
    return acc


def _k2_body(x_ref, h_ref, w_ref, o_ref):
    k = pl.program_id(1)
    w = w_ref[...]
    xr = x_ref[0].reshape(1000, 2, 512)
    e = xr[:, 0, :]
    o = xr[:, 1, :]
    nxt = jnp.where(k < 3, h_ref[0, 0:1, :], jnp.zeros((1, 512), jnp.float32))
    e1 = jnp.concatenate([e[1:], nxt], axis=0)
    acc = jnp.dot(e, w[0], preferred_element_type=jnp.float32)
    acc = acc + jnp.dot(o, w[1], preferred_element_type=jnp.float32)
    acc = acc + jnp.dot(e1, w[2], preferred_element_type=jnp.float32)
    o_ref[0] = _gelu(acc)


def _k3_body(x_ref, w2_ref, w3_ref, w4_ref, w5_ref, w6_ref, lns_ref, lnb_ref,
             pq_ref, pqb_ref, me_ref, fw_ref, fb_ref, oh_ref, o_ref):
    x = _gelu(_conv_s2(x_ref[0], w2_ref[...], 2000, 3))
    x = _gelu(_conv_s2(x, w3_ref[...], 1000, 3))
    x = _gelu(_conv_s2(x, w4_ref[...], 500, 3))
    x = _gelu(_conv_s2(x, w5_ref[...], 250, 2))
    x = _gelu(_conv_s2(x, w6_ref[...], 125, 2))
    m = jnp.mean(x, axis=-1, keepdims=True)
    d = x - m
    v = jnp.mean(d * d, axis=-1, keepdims=True)
    xl = d * lax.rsqrt(v + 1e-5) * lns_ref[...] + lnb_ref[...]
    y = jnp.dot(xl, pq_ref[...], preferred_element_type=jnp.float32) + pqb_ref[...]
    xv = jnp.dot(me_ref[...], fw_ref[...], preferred_element_type=jnp.float32) + fb_ref[...]
    xn = xv / (jnp.sqrt(jnp.sum(xv * xv)) + 1e-8)
    yn = y / (jnp.sqrt(jnp.sum(y * y, axis=-1, keepdims=True)) + 1e-8)
    l = jnp.sum(yn * xn, axis=-1) / _TEMP
    lp = jnp.concatenate([l, jnp.zeros((3,), jnp.float32)]).reshape(1, 128)
    sel = jnp.sum(oh_ref[0] * lp, axis=-1)
    o_ref[0, 0] = jnp.concatenate([sel, jnp.zeros((4,), jnp.float32)])


def _sc_gather(sel_table, flatidx):
    info = plsc.get_sparse_core_info()
    nw = info.num_cores * info.num_subcores
    ch = _NPAD // nw
    mesh = plsc.VectorSubcoreMesh(core_axis_name="c", subcore_axis_name="s")

    @functools.partial(
        pl.kernel, mesh=mesh,
        out_type=jax.ShapeDtypeStruct((_NPAD,), jnp.float32),
        compiler_params=pltpu.CompilerParams(needs_layout_passes=False),
        scratch_types=[
            pltpu.VMEM((128,), jnp.float32),
            pltpu.VMEM((ch,), jnp.int32),
            pltpu.VMEM((ch,), jnp.float32),
        ])
    def k(tab_hbm, idx_hbm, out_hbm, tab_v, idx_v, out_v):
        wid = lax.axis_index("s") * info.num_cores + lax.axis_index("c")
        base = wid * ch
        pltpu.sync_copy(tab_hbm, tab_v)
        pltpu.sync_copy(idx_hbm.at[pl.ds(base, ch)], idx_v)
        for j in range(ch // 16):
            jdx = idx_v[pl.ds(j * 16, 16)]
            out_v[pl.ds(j * 16, 16)] = plsc.load_gather(tab_v, [jdx])
        pltpu.sync_copy(out_v, out_hbm.at[pl.ds(base, ch)])

    return k(sel_table, flatidx)


def kernel(source, masked_pos, conv_w0, conv_w1, conv_w2, conv_w3, conv_w4,
           conv_w5, conv_w6, gn_scale, gn_bias, ln_scale, ln_bias, proj_w,
           proj_b, mask_emb, projq_w, projq_b, final_w, final_b):
    f32 = jnp.float32
    srcp = jnp.pad(source, ((0, 0), (0, 10)))
    r5 = srcp[:, :40005].reshape(_B, 8001, 5)
    p_all = jnp.concatenate([r5[:, :8000], r5[:, 1:8001]], axis=2)
    w0r = conv_w0.reshape(10, 512)

    g0 = pl.pallas_call(
        _k1_body,
        grid=(_B,),
        in_specs=[
            pl.BlockSpec((1, 8000, 10), lambda b: (b, 0, 0)),
            pl.BlockSpec((10, 512), lambda b: (0, 0)),
            pl.BlockSpec((1, 512), lambda b: (0, 0)),
            pl.BlockSpec((1, 512), lambda b: (0, 0)),
        ],
        out_specs=pl.BlockSpec((1, 8000, 512), lambda b: (b, 0, 0)),
        out_shape=jax.ShapeDtypeStruct((_B, 8000, 512), f32),
    )(p_all, w0r, gn_scale.reshape(1, 512), gn_bias.reshape(1, 512))

    g1 = pl.pallas_call(
        _k2_body,
        grid=(_B, 4),
        in_specs=[
            pl.BlockSpec((1, 2000, 512), lambda b, k: (b, k, 0)),
            pl.BlockSpec((1, 8, 512),
                         lambda b, k: (b, jnp.minimum(250 * (k + 1), 999), 0)),
            pl.BlockSpec((3, 512, 512), lambda b, k: (0, 0, 0)),
        ],
        out_specs=pl.BlockSpec((1, 1000, 512), lambda b, k: (b, k, 0)),
        out_shape=jax.ShapeDtypeStruct((_B, 4000, 512), f32),
    )(g0, g0, conv_w1)

    l0 = pl.pallas_call(
        _k3_body,
        grid=(_B,),
        in_specs=[
            pl.BlockSpec((1, 4000, 512), lambda b: (b, 0, 0)),
            pl.BlockSpec((3, 512, 512), lambda b: (0, 0, 0)),
            pl.BlockSpec((3, 512, 512), lambda b: (0, 0, 0)),
            pl.BlockSpec((3, 512, 512), lambda b: (0, 0, 0)),
            pl.BlockSpec((2, 512, 512), lambda b: (0, 0, 0)),
            pl.BlockSpec((2, 512, 512), lambda b: (0, 0, 0)),
            pl.BlockSpec((1, 512), lambda b: (0, 0)),
            pl.BlockSpec((1, 512), lambda b: (0, 0)),
            pl.BlockSpec((512, 256), lambda b: (0, 0)),
            pl.BlockSpec((1, 256), lambda b: (0, 0)),
            pl.BlockSpec((1, 768), lambda b: (0, 0)),
            pl.BlockSpec((768, 256), lambda b: (0, 0)),
            pl.BlockSpec((1, 256), lambda b: (0, 0)),
            pl.BlockSpec((1, 60, 128), lambda b: (b, 0, 0)),
        ],
        out_specs=pl.BlockSpec((1, 1, 64), lambda b: (b, 0, 0)),
        out_shape=jax.ShapeDtypeStruct((_B, 1, 64), f32),
    )(g1, conv_w2, conv_w3, conv_w4, conv_w5, conv_w6,
      ln_scale.reshape(1, 512), ln_bias.reshape(1, 512),
      projq_w, projq_b.reshape(1, 256),
      mask_emb.reshape(1, 768), final_w, final_b.reshape(1, 256),
      jax.nn.one_hot(masked_pos, 128, dtype=f32))

    flat = _sc_gather(l0.reshape(_B * 64), _neg_index_pattern())
    return flat[:_NOUT].reshape(1 + _NNEG, _B, _M)
